# baseline (speedup 1.0000x reference)
import functools

import jax
import jax.numpy as jnp
from jax import lax
from jax.experimental import pallas as pl
from jax.experimental.pallas import tpu as pltpu
from jax.experimental.pallas import tpu_sc as plsc

_V = 1000000
_D = 32
_B = 16384
_F = 26
_NC = 2
_NS = 16
_NW = _NC * _NS
_NBLK = _V // 128
_PW1 = (_NBLK + _NW - 1) // _NW
_TAIL = _V - _NBLK * 128
_PB = _B // _NW
_NBL = _B // 128


def _k1_transpose(tbl_t):
    mesh = plsc.VectorSubcoreMesh(core_axis_name="c", subcore_axis_name="s")

    @functools.partial(
        pl.kernel, mesh=mesh,
        out_type=jax.ShapeDtypeStruct((_V * _D // 128, 128), jnp.float32),
        scratch_types=[
            pltpu.VMEM((32, 128), jnp.float32),
            pltpu.VMEM((32, 128), jnp.float32),
            pltpu.VMEM((32, _TAIL), jnp.float32),
            pltpu.VMEM((_TAIL * _D // 128, 128), jnp.float32),
        ],
        compiler_params=pltpu.CompilerParams(
            use_tc_tiling_on_sc=True, needs_layout_passes=False),
    )
    def k(tbl_ref, scr_ref, tile_v, rot_v, tile2_v, rot2_v):
        w = lax.axis_index("s") * _NC + lax.axis_index("c")
        iota = lax.iota(jnp.int32, 16)
        half = iota + 16

        def body(t, carry):
            l = jnp.minimum(w * _PW1 + t, _NBLK - 1)
            pltpu.sync_copy(tbl_ref.at[:, pl.ds(l * 128, 128)], tile_v)
            for q in range(128):
                qv = jnp.full((16,), q, jnp.int32)
                lo = plsc.load_gather(tile_v, [iota, qv])
                hi = plsc.load_gather(tile_v, [half, qv])
                rot_v[q // 4, pl.ds((q % 4) * 32, 16)] = lo
                rot_v[q // 4, pl.ds((q % 4) * 32 + 16, 16)] = hi
            pltpu.sync_copy(rot_v, scr_ref.at[pl.ds(l * 32, 32)])
            return carry

        lax.fori_loop(0, _PW1, body, 0)

        @pl.when(w == _NW - 1)
        def _():
            pltpu.sync_copy(tbl_ref.at[:, pl.ds(_NBLK * 128, _TAIL)], tile2_v)
            for q in range(_TAIL):
                qv = jnp.full((16,), q, jnp.int32)
                lo = plsc.load_gather(tile2_v, [iota, qv])
                hi = plsc.load_gather(tile2_v, [half, qv])
                p = q * _D
                rot2_v[p // 128, pl.ds(p % 128, 16)] = lo
                rot2_v[p // 128, pl.ds(p % 128 + 16, 16)] = hi
            pltpu.sync_copy(
                rot2_v, scr_ref.at[pl.ds(_NBLK * 32, _TAIL * _D // 128)])

    return k(tbl_t)


def _k2_gather(tbl_lin, agent_ids):
    mesh = plsc.VectorSubcoreMesh(core_axis_name="c", subcore_axis_name="s")

    @functools.partial(
        pl.kernel, mesh=mesh,
        out_type=jax.ShapeDtypeStruct((_F, 4, _NBL, 8, 128), jnp.float32),
        scratch_types=[
            pltpu.VMEM((_PB, _F), jnp.int32),
            pltpu.VMEM((_F, _PB), jnp.int32),
            pltpu.VMEM((_PB, _D), jnp.float32),
            pltpu.VMEM((4, 8, 128), jnp.float32),
            pltpu.SemaphoreType.DMA,
        ],
        compiler_params=pltpu.CompilerParams(
            use_tc_tiling_on_sc=False, needs_layout_passes=False),
    )
    def k(tbl_ref, idx_ref, out_ref, idx_v, idxt_v, rows_v, obuf_v, sem):
        w = lax.axis_index("s") * _NC + lax.axis_index("c")
        b0 = w * _PB
        iota = lax.iota(jnp.int32, 16)
        pltpu.sync_copy(idx_ref.at[pl.ds(b0, _PB)], idx_v)

        def tbody(f, carry):
            fv = jnp.full((16,), f, jnp.int32)
            for kk in range(_PB // 16):
                v = plsc.load_gather(idx_v, [iota + kk * 16, fv])
                idxt_v[f, pl.ds(kk * 16, 16)] = v
            return carry

        lax.fori_loop(0, _F, tbody, 0)

        def fbody(f, carry):
            for c in range(_PB // 128):
                pltpu.async_copy(
                    tbl_ref.at[idxt_v.at[f, pl.ds(c * 128, 128)]],
                    rows_v.at[pl.ds(c * 128, 128)], sem)
            for c in range(_PB // 128):
                pltpu.make_async_copy(
                    tbl_ref.at[idxt_v.at[0, pl.ds(0, 128)]],
                    rows_v.at[pl.ds(c * 128, 128)], sem).wait()

            def esbody(es, carry2):
                for e8 in range(8):
                    cv = jnp.full((16,), es * 8 + e8, jnp.int32)
                    for j in range(4):
                        for kk in range(8):
                            v = plsc.load_gather(
                                rows_v, [iota + (j * 128 + kk * 16), cv])
                            obuf_v[j, e8, pl.ds(kk * 16, 16)] = v
                pltpu.sync_copy(
                    obuf_v, out_ref.at[f, es, pl.ds(4 * w, 4)])
                return carry2

            lax.fori_loop(0, 4, esbody, 0)
            return carry

        lax.fori_loop(0, _F, fbody, 0)

    return k(tbl_lin, agent_ids)


def kernel(agent_ids, table):
    scr = _k1_transpose(table.T)
    tbl_lin = scr.reshape(_V, _D)
    o5 = _k2_gather(tbl_lin, agent_ids)
    return o5.transpose(2, 4, 0, 1, 3).reshape(_B, _F, _D)


# speedup vs baseline: 1.4903x; 1.4903x over previous
import functools

import jax
import jax.numpy as jnp
from jax import lax
from jax.experimental import pallas as pl
from jax.experimental.pallas import tpu as pltpu
from jax.experimental.pallas import tpu_sc as plsc

_V = 1000000
_D = 32
_B = 16384
_F = 26
_NC = 2
_NS = 16
_NW = _NC * _NS
_NBLK = _V // 128
_PW1 = (_NBLK + _NW - 1) // _NW
_TAIL = _V - _NBLK * 128
_PB = _B // _NW
_NBL = _B // 128


def _k1_transpose(tbl_t):
    mesh = plsc.VectorSubcoreMesh(core_axis_name="c", subcore_axis_name="s")

    @functools.partial(
        pl.kernel, mesh=mesh,
        out_type=jax.ShapeDtypeStruct((_V * _D,), jnp.float32),
        scratch_types=[
            pltpu.VMEM((32, 128), jnp.float32),
            pltpu.VMEM((32, 128), jnp.float32),
            pltpu.VMEM((4096,), jnp.float32),
            pltpu.VMEM((4096,), jnp.float32),
            pltpu.VMEM((32, _TAIL), jnp.float32),
            pltpu.VMEM((_TAIL * _D,), jnp.float32),
            pltpu.SemaphoreType.DMA,
            pltpu.SemaphoreType.DMA,
            pltpu.SemaphoreType.DMA,
            pltpu.SemaphoreType.DMA,
        ],
        compiler_params=pltpu.CompilerParams(
            use_tc_tiling_on_sc=True, needs_layout_passes=False),
    )
    def k(tbl_ref, scr_ref, tile0, tile1, rot0, rot1,
          tile2_v, rot2_v, isem0, isem1, osem0, osem1):
        w = lax.axis_index("s") * _NC + lax.axis_index("c")
        iota = lax.iota(jnp.int32, 16)
        qstep = iota * _D
        tiles = (tile0, tile1)
        rots = (rot0, rot1)
        isems = (isem0, isem1)
        osems = (osem0, osem1)

        def blk(t):
            return jnp.minimum(w * _PW1 + t, _NBLK - 1)

        def issue_in(t, par):
            pltpu.async_copy(
                tbl_ref.at[:, pl.ds(blk(t) * 128, 128)], tiles[par],
                isems[par])

        def wait_in(par):
            pltpu.make_async_copy(
                tbl_ref.at[:, pl.ds(0, 128)], tiles[par], isems[par]).wait()

        def wait_out(par):
            pltpu.make_async_copy(
                rots[par], scr_ref.at[pl.ds(0, 4096)], osems[par]).wait()

        issue_in(0, 0)
        issue_in(1, 1)

        def step(t, par):
            wait_in(par)

            @pl.when(t >= 2)
            def _():
                wait_out(par)

            tile, rot = tiles[par], rots[par]
            for e in range(32):
                lo = tile[e, pl.ds(0, 16)]
                hi = tile[e, pl.ds(16, 16)]
                plsc.store_scatter(rot, [qstep + e], lo)
                plsc.store_scatter(rot, [qstep + (16 * _D + e)], hi)
                for kk in range(1, 4):
                    lo = tile[e, pl.ds(kk * 32, 16)]
                    hi = tile[e, pl.ds(kk * 32 + 16, 16)]
                    plsc.store_scatter(rot, [qstep + (kk * 32 * _D + e)], lo)
                    plsc.store_scatter(
                        rot, [qstep + ((kk * 32 + 16) * _D + e)], hi)
            issue_in(t + 2, par)
            pltpu.async_copy(
                rot, scr_ref.at[pl.ds(blk(t) * 4096, 4096)], osems[par])

        def body2(u, carry):
            step(2 * u, 0)
            step(2 * u + 1, 1)
            return carry

        lax.fori_loop(0, (_PW1 + 1) // 2, body2, 0)
        wait_in(0)
        wait_in(1)
        wait_out(0)
        wait_out(1)

        @pl.when(w == _NW - 1)
        def _():
            pltpu.sync_copy(tbl_ref.at[:, pl.ds(_NBLK * 128, _TAIL)], tile2_v)
            for e in range(32):
                for kk in range(_TAIL // 16):
                    v = tile2_v[e, pl.ds(kk * 16, 16)]
                    plsc.store_scatter(
                        rot2_v, [qstep + (kk * 16 * _D + e)], v)
            pltpu.sync_copy(
                rot2_v, scr_ref.at[pl.ds(_NBLK * 128 * _D, _TAIL * _D)])

    return k(tbl_t)


def _k2_gather(tbl_lin, agent_ids):
    mesh = plsc.VectorSubcoreMesh(core_axis_name="c", subcore_axis_name="s")

    @functools.partial(
        pl.kernel, mesh=mesh,
        out_type=jax.ShapeDtypeStruct((_F * 4 * _NBL * 8 * 128,), jnp.float32),
        scratch_types=[
            pltpu.VMEM((_PB, _F), jnp.int32),
            pltpu.VMEM((_F, _PB), jnp.int32),
            pltpu.VMEM((_PB, _D), jnp.float32),
            pltpu.VMEM((_PB, _D), jnp.float32),
            pltpu.VMEM((16384,), jnp.float32),
            pltpu.VMEM((16384,), jnp.float32),
            pltpu.SemaphoreType.DMA,
            pltpu.SemaphoreType.DMA,
            pltpu.SemaphoreType.DMA,
            pltpu.SemaphoreType.DMA,
        ],
        compiler_params=pltpu.CompilerParams(
            use_tc_tiling_on_sc=False, needs_layout_passes=False),
    )
    def k(tbl_ref, idx_ref, out_ref, idx_v, idxt_v, rows0, rows1,
          ob0, ob1, gsem0, gsem1, osem0, osem1):
        w = lax.axis_index("s") * _NC + lax.axis_index("c")
        iota = lax.iota(jnp.int32, 16)
        rows = (rows0, rows1)
        obufs = (ob0, ob1)
        gsems = (gsem0, gsem1)
        osems = (osem0, osem1)

        pltpu.sync_copy(idx_ref.at[pl.ds(w * _PB, _PB)], idx_v)

        def tbody(f, carry):
            fv = jnp.full((16,), f, jnp.int32)
            for kk in range(_PB // 16):
                v = plsc.load_gather(idx_v, [iota + kk * 16, fv])
                idxt_v[f, pl.ds(kk * 16, 16)] = v
            return carry

        lax.fori_loop(0, _F, tbody, 0)

        elo = (iota // 8) * 4096 + (iota % 8) * 128
        ehi = ((iota + 16) // 8) * 4096 + (iota % 8) * 128

        def issue_gathers(f, par):
            fc = jnp.minimum(f, _F - 1)
            for c in range(_PB // 128):
                pltpu.async_copy(
                    tbl_ref.at[idxt_v.at[fc, pl.ds(c * 128, 128)]],
                    rows[par].at[pl.ds(c * 128, 128)], gsems[par])

        def wait_gathers(par):
            for c in range(_PB // 128):
                pltpu.make_async_copy(
                    tbl_ref.at[idxt_v.at[0, pl.ds(0, 128)]],
                    rows[par].at[pl.ds(c * 128, 128)], gsems[par]).wait()

        def wait_outs(par):
            for es in range(4):
                pltpu.make_async_copy(
                    obufs[par].at[pl.ds(es * 4096, 4096)],
                    out_ref.at[pl.ds(0, 4096)], osems[par]).wait()

        issue_gathers(0, 0)
        issue_gathers(1, 1)

        def step(f, par):
            wait_gathers(par)

            @pl.when(f >= 2)
            def _():
                wait_outs(par)

            rv, ob = rows[par], obufs[par]

            def jbody(j, carry2):
                joff = j * 1024
                for kk in range(128):
                    b = j * 128 + kk
                    lo = rv[b, pl.ds(0, 16)]
                    hi = rv[b, pl.ds(16, 16)]
                    plsc.store_scatter(ob, [elo + (joff + kk)], lo)
                    plsc.store_scatter(ob, [ehi + (joff + kk)], hi)
                return carry2

            lax.fori_loop(0, 4, jbody, 0)
            issue_gathers(f + 2, par)
            for es in range(4):
                off = ((f * 4 + es) * _NBL + 4 * w) * 1024
                pltpu.async_copy(
                    ob.at[pl.ds(es * 4096, 4096)],
                    out_ref.at[pl.ds(off, 4096)], osems[par])
            return None

        def body2(u, carry):
            step(2 * u, 0)
            step(2 * u + 1, 1)
            return carry

        lax.fori_loop(0, _F // 2, body2, 0)
        wait_gathers(0)
        wait_gathers(1)
        wait_outs(0)
        wait_outs(1)

    return k(tbl_lin, agent_ids)


def kernel(agent_ids, table):
    scr = _k1_transpose(table.T)
    tbl_lin = scr.reshape(_V, _D)
    o5 = _k2_gather(tbl_lin, agent_ids)
    return (o5.reshape(_F, 4, _NBL, 8, 128)
            .transpose(2, 4, 0, 1, 3).reshape(_B, _F, _D))


# speedup vs baseline: 1.7192x; 1.1536x over previous
import functools

import jax
import jax.numpy as jnp
from jax import lax
from jax.experimental import pallas as pl
from jax.experimental.pallas import tpu as pltpu
from jax.experimental.pallas import tpu_sc as plsc

_V = 1000000
_D = 32
_B = 16384
_F = 26
_NC = 2
_NS = 16
_NW = _NC * _NS
_NBLK = _V // 128
_PW1 = (_NBLK + _NW - 1) // _NW
_TAIL = _V - _NBLK * 128
_PB = _B // _NW
_NBL = _B // 128
_TP = 133
_TP2 = 69
_OP = 133


def _k1_transpose(tbl_t):
    mesh = plsc.VectorSubcoreMesh(core_axis_name="c", subcore_axis_name="s")

    @functools.partial(
        pl.kernel, mesh=mesh,
        out_type=jax.ShapeDtypeStruct((_V * _D,), jnp.float32),
        scratch_types=[
            pltpu.VMEM((32, _TP), jnp.float32),
            pltpu.VMEM((32, _TP), jnp.float32),
            pltpu.VMEM((4096,), jnp.float32),
            pltpu.VMEM((4096,), jnp.float32),
            pltpu.VMEM((32, _TAIL), jnp.float32),
            pltpu.VMEM((_TAIL * _D,), jnp.float32),
            pltpu.SemaphoreType.DMA,
            pltpu.SemaphoreType.DMA,
            pltpu.SemaphoreType.DMA,
            pltpu.SemaphoreType.DMA,
        ],
        compiler_params=pltpu.CompilerParams(
            use_tc_tiling_on_sc=True, needs_layout_passes=False),
    )
    def k(tbl_ref, scr_ref, tile0, tile1, rot0, rot1,
          tile2_v, rot2_v, isem0, isem1, osem0, osem1):
        w = lax.axis_index("s") * _NC + lax.axis_index("c")
        iota = lax.iota(jnp.int32, 16)
        half = iota + 16
        tiles = (tile0, tile1)
        rots = (rot0, rot1)
        isems = (isem0, isem1)
        osems = (osem0, osem1)

        def blk(t):
            return jnp.minimum(w * _PW1 + t, _NBLK - 1)

        def issue_in(t, par):
            pltpu.async_copy(
                tbl_ref.at[:, pl.ds(blk(t) * 128, 128)],
                tiles[par].at[:, pl.ds(0, 128)], isems[par])

        def wait_in(par):
            pltpu.make_async_copy(
                tbl_ref.at[:, pl.ds(0, 128)],
                tiles[par].at[:, pl.ds(0, 128)], isems[par]).wait()

        def wait_out(par):
            pltpu.make_async_copy(
                rots[par], scr_ref.at[pl.ds(0, 4096)], osems[par]).wait()

        issue_in(0, 0)
        issue_in(1, 1)

        def step(t, par):
            wait_in(par)

            @pl.when(t >= 2)
            def _():
                wait_out(par)

            tile, rot = tiles[par], rots[par]

            def qbody(qi, c2):
                for u in range(8):
                    q = qi * 8 + u
                    qv = jnp.zeros((16,), jnp.int32) + q
                    lo = plsc.load_gather(tile, [iota, qv])
                    hi = plsc.load_gather(tile, [half, qv])
                    rot[pl.ds(q * _D, 16)] = lo
                    rot[pl.ds(q * _D + 16, 16)] = hi
                return c2

            lax.fori_loop(0, 16, qbody, 0)
            issue_in(t + 2, par)
            pltpu.async_copy(
                rot, scr_ref.at[pl.ds(blk(t) * 4096, 4096)], osems[par])

        def body2(u, carry):
            step(2 * u, 0)
            step(2 * u + 1, 1)
            return carry

        lax.fori_loop(0, (_PW1 + 1) // 2, body2, 0)
        wait_in(0)
        wait_in(1)
        wait_out(0)
        wait_out(1)

        @pl.when(w == _NW - 1)
        def _():
            pltpu.sync_copy(
                tbl_ref.at[:, pl.ds(_NBLK * 128, _TAIL)], tile2_v)
            def q2body(qi, c2):
                for u in range(8):
                    q = qi * 8 + u
                    qv = jnp.zeros((16,), jnp.int32) + q
                    lo = plsc.load_gather(tile2_v, [iota, qv])
                    hi = plsc.load_gather(tile2_v, [half, qv])
                    rot2_v[pl.ds(q * _D, 16)] = lo
                    rot2_v[pl.ds(q * _D + 16, 16)] = hi
                return c2

            lax.fori_loop(0, _TAIL // 8, q2body, 0)
            pltpu.sync_copy(
                rot2_v, scr_ref.at[pl.ds(_NBLK * 128 * _D, _TAIL * _D)])

    return k(tbl_t)


def _k2_gather(tbl_lin, agent_ids):
    mesh = plsc.VectorSubcoreMesh(core_axis_name="c", subcore_axis_name="s")

    @functools.partial(
        pl.kernel, mesh=mesh,
        out_type=jax.ShapeDtypeStruct((_F * 4 * _NBL * 8, 128), jnp.float32),
        scratch_types=[
            pltpu.VMEM((_PB, _F), jnp.int32),
            pltpu.VMEM((_F, _PB), jnp.int32),
            pltpu.VMEM((_PB, _D), jnp.float32),
            pltpu.VMEM((_PB, _D), jnp.float32),
            pltpu.VMEM((128, _OP), jnp.float32),
            pltpu.VMEM((128, _OP), jnp.float32),
            pltpu.SemaphoreType.DMA,
            pltpu.SemaphoreType.DMA,
            pltpu.SemaphoreType.DMA,
            pltpu.SemaphoreType.DMA,
        ],
        compiler_params=pltpu.CompilerParams(
            use_tc_tiling_on_sc=False, needs_layout_passes=False),
    )
    def k(tbl_ref, idx_ref, out_ref, idx_v, idxt_v, rows0, rows1,
          ob0, ob1, gsem0, gsem1, osem0, osem1):
        w = lax.axis_index("s") * _NC + lax.axis_index("c")
        iota = lax.iota(jnp.int32, 16)
        rows = (rows0, rows1)
        obufs = (ob0, ob1)
        gsems = (gsem0, gsem1)
        osems = (osem0, osem1)

        pltpu.sync_copy(idx_ref.at[pl.ds(w * _PB, _PB)], idx_v)

        def tbody(f, carry):
            fv = jnp.full((16,), f, jnp.int32)
            for kk in range(_PB // 16):
                v = plsc.load_gather(idx_v, [iota + kk * 16, fv])
                idxt_v[f, pl.ds(kk * 16, 16)] = v
            return carry

        lax.fori_loop(0, _F, tbody, 0)

        elo = (iota // 8) * 32 + (iota % 8)
        ehi = elo + 64

        def issue_gathers(f, par):
            fc = jnp.minimum(f, _F - 1)
            for c in range(_PB // 128):
                pltpu.async_copy(
                    tbl_ref.at[idxt_v.at[fc, pl.ds(c * 128, 128)]],
                    rows[par].at[pl.ds(c * 128, 128)], gsems[par])

        def wait_gathers(par):
            for c in range(_PB // 128):
                pltpu.make_async_copy(
                    tbl_ref.at[idxt_v.at[0, pl.ds(0, 128)]],
                    rows[par].at[pl.ds(c * 128, 128)], gsems[par]).wait()

        def wait_outs(par):
            for es in range(4):
                pltpu.make_async_copy(
                    obufs[par].at[pl.ds(es * 32, 32), pl.ds(0, 128)],
                    out_ref.at[pl.ds(0, 32)], osems[par]).wait()

        issue_gathers(0, 0)
        issue_gathers(1, 1)

        def step(f, par):
            wait_gathers(par)

            @pl.when(f >= 2)
            def _():
                wait_outs(par)

            rv, ob = rows[par], obufs[par]

            def jbody(j, carry2):
                rlo = elo + j * 8
                rhi = ehi + j * 8

                def kbody(ki, c3):
                    for u in range(8):
                        kk = ki * 8 + u
                        b = j * 128 + kk
                        kv = jnp.zeros((16,), jnp.int32) + kk
                        lo = rv[b, pl.ds(0, 16)]
                        hi = rv[b, pl.ds(16, 16)]
                        plsc.store_scatter(ob, [rlo, kv], lo)
                        plsc.store_scatter(ob, [rhi, kv], hi)
                    return c3

                lax.fori_loop(0, 16, kbody, 0)
                return carry2

            lax.fori_loop(0, 4, jbody, 0)
            issue_gathers(f + 2, par)
            for es in range(4):
                roff = (f * 4 + es) * 1024 + 32 * w
                pltpu.async_copy(
                    ob.at[pl.ds(es * 32, 32), pl.ds(0, 128)],
                    out_ref.at[pl.ds(roff, 32)], osems[par])
            return None

        def body2(u, carry):
            step(2 * u, 0)
            step(2 * u + 1, 1)
            return carry

        lax.fori_loop(0, _F // 2, body2, 0)
        wait_gathers(0)
        wait_gathers(1)
        wait_outs(0)
        wait_outs(1)

    return k(tbl_lin, agent_ids)


def kernel(agent_ids, table):
    scr = _k1_transpose(table.T)
    tbl_lin = scr.reshape(_V, _D)
    o2 = _k2_gather(tbl_lin, agent_ids)
    return (o2.reshape(_F, 4, _NBL, 8, 128)
            .transpose(2, 4, 0, 1, 3).reshape(_B, _F, _D))


# speedup vs baseline: 4.3001x; 2.5012x over previous
import functools

import jax
import jax.numpy as jnp
from jax import lax
from jax.experimental import pallas as pl
from jax.experimental.pallas import tpu as pltpu
from jax.experimental.pallas import tpu_sc as plsc

_V = 1000000
_D = 32
_B = 16384
_F = 26
_NC = 2
_NS = 16
_NW = _NC * _NS
_NBLK = _V // 128
_PW1 = (_NBLK + _NW - 1) // _NW
_TAIL = _V - _NBLK * 128
_PB = _B // _NW
_NBL = _B // 128
_TP = 133
_TP2 = 69
_OP = 133


def _k1_transpose(tbl_t):
    mesh = plsc.VectorSubcoreMesh(core_axis_name="c", subcore_axis_name="s")

    @functools.partial(
        pl.kernel, mesh=mesh,
        out_type=jax.ShapeDtypeStruct((_V * _D,), jnp.float32),
        scratch_types=[
            pltpu.VMEM((32, 128), jnp.float32),
            pltpu.VMEM((32, 128), jnp.float32),
            pltpu.VMEM((4096,), jnp.float32),
            pltpu.VMEM((4096,), jnp.float32),
            pltpu.VMEM((32, _TAIL), jnp.float32),
            pltpu.VMEM((_TAIL * _D,), jnp.float32),
            pltpu.SemaphoreType.DMA,
            pltpu.SemaphoreType.DMA,
            pltpu.SemaphoreType.DMA,
            pltpu.SemaphoreType.DMA,
        ],
        compiler_params=pltpu.CompilerParams(
            use_tc_tiling_on_sc=True, needs_layout_passes=False),
    )
    def k(tbl_ref, scr_ref, tile0, tile1, rot0, rot1,
          tile2_v, rot2_v, isem0, isem1, osem0, osem1):
        w = lax.axis_index("s") * _NC + lax.axis_index("c")
        iota = lax.iota(jnp.int32, 16)
        half = iota + 16
        tiles = (tile0, tile1)
        rots = (rot0, rot1)
        isems = (isem0, isem1)
        osems = (osem0, osem1)

        def blk(t):
            return jnp.minimum(w * _PW1 + t, _NBLK - 1)

        def issue_in(t, par):
            pltpu.async_copy(
                tbl_ref.at[:, pl.ds(blk(t) * 128, 128)], tiles[par],
                isems[par])

        def wait_in(par):
            pltpu.make_async_copy(
                tbl_ref.at[:, pl.ds(0, 128)], tiles[par], isems[par]).wait()

        def wait_out(par):
            pltpu.make_async_copy(
                rots[par], scr_ref.at[pl.ds(0, 4096)], osems[par]).wait()

        issue_in(0, 0)
        issue_in(1, 1)

        def step(t, par):
            wait_in(par)

            @pl.when(t >= 2)
            def _():
                wait_out(par)

            tile, rot = tiles[par], rots[par]

            def qbody(qi, c2):
                for u in range(8):
                    q0 = qi * 8 + u
                    qmod = (iota + q0) & 127
                    lo = plsc.load_gather(tile, [iota, qmod])
                    hi = plsc.load_gather(tile, [half, qmod])
                    oaddr = qmod * _D + iota
                    plsc.store_scatter(rot, [oaddr], lo)
                    plsc.store_scatter(rot, [oaddr + 16], hi)
                return c2

            lax.fori_loop(0, 16, qbody, 0)
            issue_in(t + 2, par)
            pltpu.async_copy(
                rot, scr_ref.at[pl.ds(blk(t) * 4096, 4096)], osems[par])

        def body2(u, carry):
            step(2 * u, 0)
            step(2 * u + 1, 1)
            return carry

        lax.fori_loop(0, (_PW1 + 1) // 2, body2, 0)
        wait_in(0)
        wait_in(1)
        wait_out(0)
        wait_out(1)

        @pl.when(w == _NW - 1)
        def _():
            pltpu.sync_copy(
                tbl_ref.at[:, pl.ds(_NBLK * 128, _TAIL)], tile2_v)
            def q2body(qi, c2):
                for u in range(8):
                    q = qi * 8 + u
                    qv = jnp.zeros((16,), jnp.int32) + q
                    lo = plsc.load_gather(tile2_v, [iota, qv])
                    hi = plsc.load_gather(tile2_v, [half, qv])
                    rot2_v[pl.ds(q * _D, 16)] = lo
                    rot2_v[pl.ds(q * _D + 16, 16)] = hi
                return c2

            lax.fori_loop(0, _TAIL // 8, q2body, 0)
            pltpu.sync_copy(
                rot2_v, scr_ref.at[pl.ds(_NBLK * 128 * _D, _TAIL * _D)])

    return k(tbl_t)


def _k2_gather(tbl_lin, agent_ids):
    mesh = plsc.VectorSubcoreMesh(core_axis_name="c", subcore_axis_name="s")

    @functools.partial(
        pl.kernel, mesh=mesh,
        out_type=jax.ShapeDtypeStruct((_F * 4 * _NBL * 8, 128), jnp.float32),
        scratch_types=[
            pltpu.VMEM((_PB, _F), jnp.int32),
            pltpu.VMEM((_F, _PB), jnp.int32),
            pltpu.VMEM((_PB, _D), jnp.float32),
            pltpu.VMEM((_PB, _D), jnp.float32),
            pltpu.VMEM((128, _OP), jnp.float32),
            pltpu.VMEM((128, _OP), jnp.float32),
            pltpu.SemaphoreType.DMA,
            pltpu.SemaphoreType.DMA,
            pltpu.SemaphoreType.DMA,
            pltpu.SemaphoreType.DMA,
        ],
        compiler_params=pltpu.CompilerParams(
            use_tc_tiling_on_sc=False, needs_layout_passes=False),
    )
    def k(tbl_ref, idx_ref, out_ref, idx_v, idxt_v, rows0, rows1,
          ob0, ob1, gsem0, gsem1, osem0, osem1):
        w = lax.axis_index("s") * _NC + lax.axis_index("c")
        iota = lax.iota(jnp.int32, 16)
        rows = (rows0, rows1)
        obufs = (ob0, ob1)
        gsems = (gsem0, gsem1)
        osems = (osem0, osem1)

        pltpu.sync_copy(idx_ref.at[pl.ds(w * _PB, _PB)], idx_v)

        def tbody(f, carry):
            fv = jnp.full((16,), f, jnp.int32)
            for kk in range(_PB // 16):
                v = plsc.load_gather(idx_v, [iota + kk * 16, fv])
                idxt_v[f, pl.ds(kk * 16, 16)] = v
            return carry

        lax.fori_loop(0, _F, tbody, 0)

        elo = (iota // 8) * 32 + (iota % 8)
        ehi = elo + 64

        def issue_gathers(f, par):
            fc = jnp.minimum(f, _F - 1)
            for c in range(_PB // 128):
                pltpu.async_copy(
                    tbl_ref.at[idxt_v.at[fc, pl.ds(c * 128, 128)]],
                    rows[par].at[pl.ds(c * 128, 128)], gsems[par])

        def wait_gathers(par):
            for c in range(_PB // 128):
                pltpu.make_async_copy(
                    tbl_ref.at[idxt_v.at[0, pl.ds(0, 128)]],
                    rows[par].at[pl.ds(c * 128, 128)], gsems[par]).wait()

        def wait_outs(par):
            for es in range(4):
                pltpu.make_async_copy(
                    obufs[par].at[pl.ds(es * 32, 32), pl.ds(0, 128)],
                    out_ref.at[pl.ds(0, 32)], osems[par]).wait()

        issue_gathers(0, 0)
        issue_gathers(1, 1)

        def step(f, par):
            wait_gathers(par)

            @pl.when(f >= 2)
            def _():
                wait_outs(par)

            rv, ob = rows[par], obufs[par]

            def jbody(j, carry2):
                rlo = elo + j * 8
                rhi = ehi + j * 8

                def kbody(ki, c3):
                    for u in range(8):
                        kk = ki * 8 + u
                        b = j * 128 + kk
                        kv = jnp.zeros((16,), jnp.int32) + kk
                        lo = rv[b, pl.ds(0, 16)]
                        hi = rv[b, pl.ds(16, 16)]
                        plsc.store_scatter(ob, [rlo, kv], lo)
                        plsc.store_scatter(ob, [rhi, kv], hi)
                    return c3

                lax.fori_loop(0, 16, kbody, 0)
                return carry2

            lax.fori_loop(0, 4, jbody, 0)
            issue_gathers(f + 2, par)
            for es in range(4):
                roff = (f * 4 + es) * 1024 + 32 * w
                pltpu.async_copy(
                    ob.at[pl.ds(es * 32, 32), pl.ds(0, 128)],
                    out_ref.at[pl.ds(roff, 32)], osems[par])
            return None

        def body2(u, carry):
            step(2 * u, 0)
            step(2 * u + 1, 1)
            return carry

        lax.fori_loop(0, _F // 2, body2, 0)
        wait_gathers(0)
        wait_gathers(1)
        wait_outs(0)
        wait_outs(1)

    return k(tbl_lin, agent_ids)


def kernel(agent_ids, table):
    scr = _k1_transpose(table.T)
    tbl_lin = scr.reshape(_V, _D)
    o2 = _k2_gather(tbl_lin, agent_ids)
    return (o2.reshape(_F, 4, _NBL, 8, 128)
            .transpose(2, 4, 0, 1, 3).reshape(_B, _F, _D))


# speedup vs baseline: 4.9551x; 1.1523x over previous
import functools

import jax
import jax.numpy as jnp
from jax import lax
from jax.experimental import pallas as pl
from jax.experimental.pallas import tpu as pltpu
from jax.experimental.pallas import tpu_sc as plsc

_V = 1000000
_D = 32
_B = 16384
_F = 26
_NC = 2
_NS = 16
_NW = _NC * _NS
_NBLK = _V // 128
_PW1 = (_NBLK + _NW - 1) // _NW
_TAIL = _V - _NBLK * 128
_PB = _B // _NW
_NBL = _B // 128
_TP = 133
_TP2 = 69
_OP = 133


def _k1_transpose(tbl_t):
    mesh = plsc.VectorSubcoreMesh(core_axis_name="c", subcore_axis_name="s")

    @functools.partial(
        pl.kernel, mesh=mesh,
        out_type=jax.ShapeDtypeStruct((_V * _D,), jnp.float32),
        scratch_types=[
            pltpu.VMEM((32, 128), jnp.float32),
            pltpu.VMEM((32, 128), jnp.float32),
            pltpu.VMEM((32, 128), jnp.float32),
            pltpu.VMEM((32, 128), jnp.float32),
            pltpu.VMEM((4096,), jnp.float32),
            pltpu.VMEM((4096,), jnp.float32),
            pltpu.VMEM((4096,), jnp.float32),
            pltpu.VMEM((4096,), jnp.float32),
            pltpu.VMEM((32, _TAIL), jnp.float32),
            pltpu.VMEM((_TAIL * _D,), jnp.float32),
            pltpu.SemaphoreType.DMA,
            pltpu.SemaphoreType.DMA,
            pltpu.SemaphoreType.DMA,
            pltpu.SemaphoreType.DMA,
            pltpu.SemaphoreType.DMA,
            pltpu.SemaphoreType.DMA,
            pltpu.SemaphoreType.DMA,
            pltpu.SemaphoreType.DMA,
        ],
        compiler_params=pltpu.CompilerParams(
            use_tc_tiling_on_sc=True, needs_layout_passes=False),
    )
    def k(tbl_ref, scr_ref, tile0, tile1, tile2, tile3, rot0, rot1, rot2,
          rot3, tile2_v, rot2_v, isem0, isem1, isem2, isem3,
          osem0, osem1, osem2, osem3):
        w = lax.axis_index("s") * _NC + lax.axis_index("c")
        iota = lax.iota(jnp.int32, 16)
        half = iota + 16
        tiles = (tile0, tile1, tile2, tile3)
        rots = (rot0, rot1, rot2, rot3)
        isems = (isem0, isem1, isem2, isem3)
        osems = (osem0, osem1, osem2, osem3)

        def blk(t):
            return jnp.minimum(w * _PW1 + t, _NBLK - 1)

        def issue_in(t, par):
            pltpu.async_copy(
                tbl_ref.at[:, pl.ds(blk(t) * 128, 128)], tiles[par],
                isems[par])

        def wait_in(par):
            pltpu.make_async_copy(
                tbl_ref.at[:, pl.ds(0, 128)], tiles[par], isems[par]).wait()

        def wait_out(par):
            pltpu.make_async_copy(
                rots[par], scr_ref.at[pl.ds(0, 4096)], osems[par]).wait()

        for p in range(4):
            issue_in(p, p)

        def step(t, par):
            wait_in(par)

            @pl.when(t >= 4)
            def _():
                wait_out(par)

            tile, rot = tiles[par], rots[par]

            def qbody(qi, c2):
                for u in range(8):
                    q0 = qi * 8 + u
                    qmod = (iota + q0) & 127
                    lo = plsc.load_gather(tile, [iota, qmod])
                    hi = plsc.load_gather(tile, [half, qmod])
                    oaddr = qmod * _D + iota
                    plsc.store_scatter(rot, [oaddr], lo)
                    plsc.store_scatter(rot, [oaddr + 16], hi)
                return c2

            lax.fori_loop(0, 16, qbody, 0)
            issue_in(t + 4, par)
            pltpu.async_copy(
                rot, scr_ref.at[pl.ds(blk(t) * 4096, 4096)], osems[par])

        def body4(u, carry):
            for p in range(4):
                step(4 * u + p, p)
            return carry

        lax.fori_loop(0, (_PW1 + 3) // 4, body4, 0)
        for p in range(4):
            wait_in(p)
            wait_out(p)

        @pl.when(w == _NW - 1)
        def _():
            pltpu.sync_copy(
                tbl_ref.at[:, pl.ds(_NBLK * 128, _TAIL)], tile2_v)
            def q2body(qi, c2):
                for u in range(8):
                    q = qi * 8 + u
                    qv = jnp.zeros((16,), jnp.int32) + q
                    lo = plsc.load_gather(tile2_v, [iota, qv])
                    hi = plsc.load_gather(tile2_v, [half, qv])
                    rot2_v[pl.ds(q * _D, 16)] = lo
                    rot2_v[pl.ds(q * _D + 16, 16)] = hi
                return c2

            lax.fori_loop(0, _TAIL // 8, q2body, 0)
            pltpu.sync_copy(
                rot2_v, scr_ref.at[pl.ds(_NBLK * 128 * _D, _TAIL * _D)])

    return k(tbl_t)


def _k2_gather(tbl_lin, agent_ids):
    mesh = plsc.VectorSubcoreMesh(core_axis_name="c", subcore_axis_name="s")

    @functools.partial(
        pl.kernel, mesh=mesh,
        out_type=jax.ShapeDtypeStruct((_F * 4 * _NBL * 8, 128), jnp.float32),
        scratch_types=[
            pltpu.VMEM((_PB, _F), jnp.int32),
            pltpu.VMEM((_F, _PB), jnp.int32),
            pltpu.VMEM((_PB, _D), jnp.float32),
            pltpu.VMEM((_PB, _D), jnp.float32),
            pltpu.VMEM((_PB, _D), jnp.float32),
            pltpu.VMEM((_PB, _D), jnp.float32),
            pltpu.VMEM((128, _OP), jnp.float32),
            pltpu.VMEM((128, _OP), jnp.float32),
            pltpu.SemaphoreType.DMA,
            pltpu.SemaphoreType.DMA,
            pltpu.SemaphoreType.DMA,
            pltpu.SemaphoreType.DMA,
            pltpu.SemaphoreType.DMA,
            pltpu.SemaphoreType.DMA,
        ],
        compiler_params=pltpu.CompilerParams(
            use_tc_tiling_on_sc=False, needs_layout_passes=False),
    )
    def k(tbl_ref, idx_ref, out_ref, idx_v, idxt_v, rows0, rows1, rows2,
          rows3, ob0, ob1, gsem0, gsem1, gsem2, gsem3, osem0, osem1):
        w = lax.axis_index("s") * _NC + lax.axis_index("c")
        iota = lax.iota(jnp.int32, 16)
        rows = (rows0, rows1, rows2, rows3)
        obufs = (ob0, ob1)
        gsems = (gsem0, gsem1, gsem2, gsem3)
        osems = (osem0, osem1)

        pltpu.sync_copy(idx_ref.at[pl.ds(w * _PB, _PB)], idx_v)

        def tbody(f, carry):
            fv = jnp.full((16,), f, jnp.int32)
            for kk in range(_PB // 16):
                v = plsc.load_gather(idx_v, [iota + kk * 16, fv])
                idxt_v[f, pl.ds(kk * 16, 16)] = v
            return carry

        lax.fori_loop(0, _F, tbody, 0)

        elo = (iota // 8) * 32 + (iota % 8)
        ehi = elo + 64

        def issue_gathers(f, par):
            fc = jnp.minimum(f, _F - 1)
            for c in range(_PB // 128):
                pltpu.async_copy(
                    tbl_ref.at[idxt_v.at[fc, pl.ds(c * 128, 128)]],
                    rows[par].at[pl.ds(c * 128, 128)], gsems[par])

        def wait_gathers(par):
            for c in range(_PB // 128):
                pltpu.make_async_copy(
                    tbl_ref.at[idxt_v.at[0, pl.ds(0, 128)]],
                    rows[par].at[pl.ds(c * 128, 128)], gsems[par]).wait()

        def wait_outs(par):
            for es in range(4):
                pltpu.make_async_copy(
                    obufs[par].at[pl.ds(es * 32, 32), pl.ds(0, 128)],
                    out_ref.at[pl.ds(0, 32)], osems[par]).wait()

        for p in range(4):
            issue_gathers(p, p)

        def step(f, par4, par2):
            wait_gathers(par4)

            @pl.when(f >= 2)
            def _():
                wait_outs(par2)

            rv, ob = rows[par4], obufs[par2]

            def jbody(j, carry2):
                rlo = elo + j * 8
                rhi = ehi + j * 8

                def kbody(ki, c3):
                    for u in range(8):
                        kk = ki * 8 + u
                        b = j * 128 + kk
                        kv = jnp.zeros((16,), jnp.int32) + kk
                        lo = rv[b, pl.ds(0, 16)]
                        hi = rv[b, pl.ds(16, 16)]
                        plsc.store_scatter(ob, [rlo, kv], lo)
                        plsc.store_scatter(ob, [rhi, kv], hi)
                    return c3

                lax.fori_loop(0, 16, kbody, 0)
                return carry2

            lax.fori_loop(0, 4, jbody, 0)
            issue_gathers(f + 4, par4)
            for es in range(4):
                roff = (f * 4 + es) * 1024 + 32 * w
                pltpu.async_copy(
                    ob.at[pl.ds(es * 32, 32), pl.ds(0, 128)],
                    out_ref.at[pl.ds(roff, 32)], osems[par2])
            return None

        def body4(u, carry):
            for p in range(4):
                f = 4 * u + p
                step(f, p, p % 2)
            return carry

        lax.fori_loop(0, _F // 4, body4, 0)
        step(jnp.int32(24), 0, 0)
        step(jnp.int32(25), 1, 1)
        for p in range(4):
            wait_gathers(p)
        wait_outs(0)
        wait_outs(1)

    return k(tbl_lin, agent_ids)


def kernel(agent_ids, table):
    scr = _k1_transpose(table.T)
    tbl_lin = scr.reshape(_V, _D)
    o2 = _k2_gather(tbl_lin, agent_ids)
    return (o2.reshape(_F, 4, _NBL, 8, 128)
            .transpose(2, 4, 0, 1, 3).reshape(_B, _F, _D))


# speedup vs baseline: 5.1317x; 1.0356x over previous
import functools

import jax
import jax.numpy as jnp
from jax import lax
from jax.experimental import pallas as pl
from jax.experimental.pallas import tpu as pltpu
from jax.experimental.pallas import tpu_sc as plsc

_V = 1000000
_D = 32
_B = 16384
_F = 26
_NC = 2
_NS = 16
_NW = _NC * _NS
_NBLK = _V // 128
_PW1 = (_NBLK + _NW - 1) // _NW
_TAIL = _V - _NBLK * 128
_PB = _B // _NW
_NBL = _B // 128
_TP = 133
_TP2 = 69
_OP = 133


def _k1_transpose(tbl_t):
    mesh = plsc.VectorSubcoreMesh(core_axis_name="c", subcore_axis_name="s")

    @functools.partial(
        pl.kernel, mesh=mesh,
        out_type=jax.ShapeDtypeStruct((_V * _D,), jnp.float32),
        scratch_types=[
            pltpu.VMEM((32, 128), jnp.float32),
            pltpu.VMEM((32, 128), jnp.float32),
            pltpu.VMEM((32, 128), jnp.float32),
            pltpu.VMEM((32, 128), jnp.float32),
            pltpu.VMEM((4096,), jnp.float32),
            pltpu.VMEM((4096,), jnp.float32),
            pltpu.VMEM((4096,), jnp.float32),
            pltpu.VMEM((4096,), jnp.float32),
            pltpu.VMEM((32, _TAIL), jnp.float32),
            pltpu.VMEM((_TAIL * _D,), jnp.float32),
            pltpu.SemaphoreType.DMA,
            pltpu.SemaphoreType.DMA,
            pltpu.SemaphoreType.DMA,
            pltpu.SemaphoreType.DMA,
            pltpu.SemaphoreType.DMA,
            pltpu.SemaphoreType.DMA,
            pltpu.SemaphoreType.DMA,
            pltpu.SemaphoreType.DMA,
        ],
        compiler_params=pltpu.CompilerParams(
            use_tc_tiling_on_sc=True, needs_layout_passes=False),
    )
    def k(tbl_ref, scr_ref, tile0, tile1, tile2, tile3, rot0, rot1, rot2,
          rot3, tile2_v, rot2_v, isem0, isem1, isem2, isem3,
          osem0, osem1, osem2, osem3):
        w = lax.axis_index("s") * _NC + lax.axis_index("c")
        iota = lax.iota(jnp.int32, 16)
        half = iota + 16
        tiles = (tile0, tile1, tile2, tile3)
        rots = (rot0, rot1, rot2, rot3)
        isems = (isem0, isem1, isem2, isem3)
        osems = (osem0, osem1, osem2, osem3)

        def blk(t):
            return jnp.minimum(w * _PW1 + t, _NBLK - 1)

        def issue_in(t, par):
            pltpu.async_copy(
                tbl_ref.at[:, pl.ds(blk(t) * 128, 128)], tiles[par],
                isems[par])

        def wait_in(par):
            pltpu.make_async_copy(
                tbl_ref.at[:, pl.ds(0, 128)], tiles[par], isems[par]).wait()

        def wait_out(par):
            pltpu.make_async_copy(
                rots[par], scr_ref.at[pl.ds(0, 4096)], osems[par]).wait()

        for p in range(4):
            issue_in(p, p)

        def step(t, par):
            wait_in(par)

            @pl.when(t >= 4)
            def _():
                wait_out(par)

            tile, rot = tiles[par], rots[par]

            def qbody(qi, c2):
                for u in range(8):
                    q0 = qi * 8 + u
                    qmod = (iota + q0) & 127
                    lo = plsc.load_gather(tile, [iota, qmod])
                    hi = plsc.load_gather(tile, [half, qmod])
                    oaddr = qmod * _D + iota
                    plsc.store_scatter(rot, [oaddr], lo)
                    plsc.store_scatter(rot, [oaddr + 16], hi)
                return c2

            lax.fori_loop(0, 16, qbody, 0)
            issue_in(t + 4, par)
            pltpu.async_copy(
                rot, scr_ref.at[pl.ds(blk(t) * 4096, 4096)], osems[par])

        def body4(u, carry):
            for p in range(4):
                step(4 * u + p, p)
            return carry

        lax.fori_loop(0, (_PW1 + 3) // 4, body4, 0)
        for p in range(4):
            wait_in(p)
            wait_out(p)

        @pl.when(w == _NW - 1)
        def _():
            pltpu.sync_copy(
                tbl_ref.at[:, pl.ds(_NBLK * 128, _TAIL)], tile2_v)
            def q2body(qi, c2):
                for u in range(8):
                    q = qi * 8 + u
                    qv = jnp.zeros((16,), jnp.int32) + q
                    lo = plsc.load_gather(tile2_v, [iota, qv])
                    hi = plsc.load_gather(tile2_v, [half, qv])
                    rot2_v[pl.ds(q * _D, 16)] = lo
                    rot2_v[pl.ds(q * _D + 16, 16)] = hi
                return c2

            lax.fori_loop(0, _TAIL // 8, q2body, 0)
            pltpu.sync_copy(
                rot2_v, scr_ref.at[pl.ds(_NBLK * 128 * _D, _TAIL * _D)])

    return k(tbl_t)


def _k2_gather(tbl_lin, agent_ids):
    mesh = plsc.VectorSubcoreMesh(core_axis_name="c", subcore_axis_name="s")

    @functools.partial(
        pl.kernel, mesh=mesh,
        out_type=jax.ShapeDtypeStruct((_F * 4 * _NBL * 8 * 128,), jnp.float32),
        scratch_types=[
            pltpu.VMEM((_PB, _F), jnp.int32),
            pltpu.VMEM((_F, _PB), jnp.int32),
            pltpu.VMEM((_PB, _D), jnp.float32),
            pltpu.VMEM((_PB, _D), jnp.float32),
            pltpu.VMEM((_PB, _D), jnp.float32),
            pltpu.VMEM((_PB, _D), jnp.float32),
            pltpu.VMEM((16384,), jnp.float32),
            pltpu.VMEM((16384,), jnp.float32),
            pltpu.SemaphoreType.DMA,
            pltpu.SemaphoreType.DMA,
            pltpu.SemaphoreType.DMA,
            pltpu.SemaphoreType.DMA,
            pltpu.SemaphoreType.DMA,
            pltpu.SemaphoreType.DMA,
        ],
        compiler_params=pltpu.CompilerParams(
            use_tc_tiling_on_sc=False, needs_layout_passes=False),
    )
    def k(tbl_ref, idx_ref, out_ref, idx_v, idxt_v, rows0, rows1, rows2,
          rows3, ob0, ob1, gsem0, gsem1, gsem2, gsem3, osem0, osem1):
        w = lax.axis_index("s") * _NC + lax.axis_index("c")
        iota = lax.iota(jnp.int32, 16)
        rows = (rows0, rows1, rows2, rows3)
        obufs = (ob0, ob1)
        gsems = (gsem0, gsem1, gsem2, gsem3)
        osems = (osem0, osem1)

        pltpu.sync_copy(idx_ref.at[pl.ds(w * _PB, _PB)], idx_v)

        def tbody(f, carry):
            fv = jnp.full((16,), f, jnp.int32)
            for kk in range(_PB // 16):
                v = plsc.load_gather(idx_v, [iota + kk * 16, fv])
                idxt_v[f, pl.ds(kk * 16, 16)] = v
            return carry

        lax.fori_loop(0, _F, tbody, 0)

        sclo = (iota // 8) * 4096 + (iota % 8) * 128
        schi = sclo + 8192

        def issue_gathers(f, par):
            fc = jnp.minimum(f, _F - 1)
            for c in range(_PB // 128):
                pltpu.async_copy(
                    tbl_ref.at[idxt_v.at[fc, pl.ds(c * 128, 128)]],
                    rows[par].at[pl.ds(c * 128, 128)], gsems[par])

        def wait_gathers(par):
            for c in range(_PB // 128):
                pltpu.make_async_copy(
                    tbl_ref.at[idxt_v.at[0, pl.ds(0, 128)]],
                    rows[par].at[pl.ds(c * 128, 128)], gsems[par]).wait()

        def wait_outs(par):
            for es in range(4):
                pltpu.make_async_copy(
                    obufs[par].at[pl.ds(es * 4096, 4096)],
                    out_ref.at[pl.ds(0, 4096)], osems[par]).wait()

        for p in range(4):
            issue_gathers(p, p)

        def step(f, par4, par2):
            wait_gathers(par4)

            @pl.when(f >= 2)
            def _():
                wait_outs(par2)

            rv, ob = rows[par4], obufs[par2]

            def jbody(j, carry2):
                jrow = j * 128
                joff = j * 1024

                def kbody(ki, c3):
                    for u in range(8):
                        kk = ki * 8 + u
                        bmod = (iota + kk * 16) & 127
                        lo = plsc.load_gather(rv, [jrow + bmod, iota])
                        hi = plsc.load_gather(rv, [jrow + bmod, iota + 16])
                        base = bmod + joff
                        plsc.store_scatter(ob, [sclo + base], lo)
                        plsc.store_scatter(ob, [schi + base], hi)
                    return c3

                lax.fori_loop(0, 16, kbody, 0)
                return carry2

            lax.fori_loop(0, 4, jbody, 0)
            issue_gathers(f + 4, par4)
            for es in range(4):
                woff = ((f * 4 + es) * 1024 + 32 * w) * 128
                pltpu.async_copy(
                    ob.at[pl.ds(es * 4096, 4096)],
                    out_ref.at[pl.ds(woff, 4096)], osems[par2])
            return None

        def body4(u, carry):
            for p in range(4):
                f = 4 * u + p
                step(f, p, p % 2)
            return carry

        lax.fori_loop(0, _F // 4, body4, 0)
        step(jnp.int32(24), 0, 0)
        step(jnp.int32(25), 1, 1)
        for p in range(4):
            wait_gathers(p)
        wait_outs(0)
        wait_outs(1)

    return k(tbl_lin, agent_ids)


def kernel(agent_ids, table):
    scr = _k1_transpose(table.T)
    tbl_lin = scr.reshape(_V, _D)
    o1 = _k2_gather(tbl_lin, agent_ids)
    return (o1.reshape(_F, 4, _NBL, 8, 128)
            .transpose(2, 4, 0, 1, 3).reshape(_B, _F, _D))
